# Initial kernel scaffold; baseline (speedup 1.0000x reference)
import jax
import jax.numpy as jnp
from jax.experimental import pallas as pl


def kernel(indices, table):
    raise NotImplementedError("write your pallas kernel here")


# speedup vs baseline: 1.4905x; 1.4905x over previous
import functools

import jax
import jax.numpy as jnp
from jax import lax
from jax.experimental import pallas as pl
from jax.experimental.pallas import tpu as pltpu
from jax.experimental.pallas import tpu_sc as plsc

VOCAB = 1000000
EMBED_DIM = 32
BATCH = 4096
SEQ = 200

_INFO = plsc.get_sparse_core_info()
_NC, _NS = _INFO.num_cores, _INFO.num_subcores
_NW = _NC * _NS
_N = BATCH * SEQ
_B_PER_W = _N // _NW
_CHUNK = 2560
_N_CHUNKS = _B_PER_W // _CHUNK


def _make_kernel():
    mesh = plsc.VectorSubcoreMesh(core_axis_name="c", subcore_axis_name="s")

    @functools.partial(
        pl.kernel,
        out_type=jax.ShapeDtypeStruct((_N, EMBED_DIM), jnp.float32),
        mesh=mesh,
        scratch_types=[
            pltpu.VMEM((_CHUNK,), jnp.int32),
            pltpu.VMEM((_CHUNK, EMBED_DIM), jnp.float32),
            pltpu.SemaphoreType.DMA,
        ],
        compiler_params=pltpu.CompilerParams(use_tc_tiling_on_sc=False),
    )
    def gather_kernel(idx_hbm, table_hbm, out_hbm, idx_v, rows_v, sem):
        wid = lax.axis_index("s") * _NC + lax.axis_index("c")
        base = wid * _B_PER_W

        def body(i, carry):
            off = base + i * _CHUNK
            pltpu.sync_copy(idx_hbm.at[pl.ds(off, _CHUNK)], idx_v)
            pltpu.async_copy(table_hbm.at[idx_v], rows_v, sem).wait()
            pltpu.sync_copy(rows_v, out_hbm.at[pl.ds(off, _CHUNK)])
            return carry

        lax.fori_loop(0, _N_CHUNKS, body, 0)

    return gather_kernel


_GATHER = _make_kernel()


def kernel(indices, table):
    flat_idx = indices.reshape(-1).astype(jnp.int32)
    out = _GATHER(flat_idx, table)
    return out.reshape(BATCH, SEQ, EMBED_DIM)


# speedup vs baseline: 1.4914x; 1.0006x over previous
import functools

import jax
import jax.numpy as jnp
from jax import lax
from jax.experimental import pallas as pl
from jax.experimental.pallas import tpu as pltpu
from jax.experimental.pallas import tpu_sc as plsc

VOCAB = 1000000
EMBED_DIM = 32
BATCH = 4096
SEQ = 200

_INFO = plsc.get_sparse_core_info()
_NC, _NS = _INFO.num_cores, _INFO.num_subcores
_NW = _NC * _NS
_N = BATCH * SEQ
_B_PER_W = _N // _NW
_CHUNK = 1600
_N_CHUNKS = _B_PER_W // _CHUNK


def _make_kernel():
    mesh = plsc.VectorSubcoreMesh(core_axis_name="c", subcore_axis_name="s")

    @functools.partial(
        pl.kernel,
        out_type=jax.ShapeDtypeStruct((_N, EMBED_DIM), jnp.float32),
        mesh=mesh,
        scratch_types=[
            pltpu.VMEM((_CHUNK,), jnp.int32),
            pltpu.VMEM((_CHUNK,), jnp.int32),
            pltpu.VMEM((_CHUNK, EMBED_DIM), jnp.float32),
            pltpu.VMEM((_CHUNK, EMBED_DIM), jnp.float32),
            pltpu.SemaphoreType.DMA,
            pltpu.SemaphoreType.DMA,
            pltpu.SemaphoreType.DMA,
            pltpu.SemaphoreType.DMA,
        ],
        compiler_params=pltpu.CompilerParams(use_tc_tiling_on_sc=False),
    )
    def gather_kernel(idx_hbm, table_hbm, out_hbm,
                      idx0, idx1, rows0, rows1, g0, g1, s0, s1):
        wid = lax.axis_index("s") * _NC + lax.axis_index("c")
        base = wid * _B_PER_W

        idx_v = [idx0, idx1]
        rows_v = [rows0, rows1]
        gsem = [g0, g1]
        ssem = [s0, s1]

        gather_h = [None, None]
        store_h = [None, None]
        prev_off = None

        for i in range(_N_CHUNKS):
            b = i % 2
            off = base + i * _CHUNK
            if store_h[b] is not None:
                store_h[b].wait()
                store_h[b] = None
            pltpu.sync_copy(idx_hbm.at[pl.ds(off, _CHUNK)], idx_v[b])
            gather_h[b] = pltpu.async_copy(table_hbm.at[idx_v[b]],
                                           rows_v[b], gsem[b])
            o = 1 - b
            if gather_h[o] is not None:
                gather_h[o].wait()
                gather_h[o] = None
                store_h[o] = pltpu.async_copy(
                    rows_v[o], out_hbm.at[pl.ds(prev_off, _CHUNK)], ssem[o])
            prev_off = off

        last = (_N_CHUNKS - 1) % 2
        gather_h[last].wait()
        pltpu.async_copy(rows_v[last],
                         out_hbm.at[pl.ds(prev_off, _CHUNK)],
                         ssem[last]).wait()
        if store_h[1 - last] is not None:
            store_h[1 - last].wait()

    return gather_kernel


_GATHER = _make_kernel()


def kernel(indices, table):
    flat_idx = indices.reshape(-1).astype(jnp.int32)
    out = _GATHER(flat_idx, table)
    return out.reshape(BATCH, SEQ, EMBED_DIM)
